# baseline (speedup 1.0000x reference)
import functools

import jax
import jax.numpy as jnp
from jax import lax
from jax.experimental import pallas as pl
from jax.experimental.pallas import tpu as pltpu
from jax.experimental.pallas import tpu_sc as plsc

N = 10000
E = 320000
G = 256
H = 128
F32 = jnp.float32

_NT = 5
_NROW = N // _NT
_EROW = 2560

_CHUNK = 64
_CPT = 160
_EPAD = 32 * _CPT * _CHUNK
_NPAD = N + 16
_RPT = 624


def _mm_bias_kernel(x_ref, w_ref, b_ref, o_ref):
    o_ref[...] = jnp.dot(x_ref[...], w_ref[...],
                         preferred_element_type=F32) + b_ref[...]


def _node_embed(x, w, b):
    return pl.pallas_call(
        _mm_bias_kernel,
        grid=(_NT,),
        in_specs=[
            pl.BlockSpec((_NROW, H), lambda i: (i, 0)),
            pl.BlockSpec((H, H), lambda i: (0, 0)),
            pl.BlockSpec((1, H), lambda i: (0, 0)),
        ],
        out_specs=pl.BlockSpec((_NROW, H), lambda i: (i, 0)),
        out_shape=jax.ShapeDtypeStruct((N, H), F32),
    )(x, w, b.reshape(1, H))


def _edge_mlp_kernel(a_ref, w1_ref, b1_ref, w2_ref, b2_ref, o_ref):
    t = jnp.maximum(jnp.dot(a_ref[...], w1_ref[...],
                            preferred_element_type=F32) + b1_ref[...], 0.0)
    o_ref[...] = jnp.dot(t, w2_ref[...],
                         preferred_element_type=F32) + b2_ref[...]


def _edge_mlp(a, w1, b1, w2, b2):
    d = a.shape[1]
    ne = a.shape[0]
    return pl.pallas_call(
        _edge_mlp_kernel,
        grid=(ne // _EROW,),
        in_specs=[
            pl.BlockSpec((_EROW, d), lambda i: (i, 0)),
            pl.BlockSpec((d, H), lambda i: (0, 0)),
            pl.BlockSpec((1, H), lambda i: (0, 0)),
            pl.BlockSpec((H, H), lambda i: (0, 0)),
            pl.BlockSpec((1, H), lambda i: (0, 0)),
        ],
        out_specs=pl.BlockSpec((_EROW, H), lambda i: (i, 0)),
        out_shape=jax.ShapeDtypeStruct((ne, H), F32),
    )(a, w1, b1.reshape(1, H), w2, b2.reshape(1, H))


def _node_layer_kernel(h_ref, p0_ref, p1_ref, w1_ref, b1_ref, w2_ref, b2_ref,
                       t_ref, stats_ref, ssum, ssq):
    i = pl.program_id(0)

    @pl.when(i == 0)
    def _():
        ssum[...] = jnp.zeros_like(ssum)
        ssq[...] = jnp.zeros_like(ssq)

    z = h_ref[...] + p0_ref[...] + p1_ref[...]
    t = jnp.maximum(jnp.dot(z, w1_ref[...],
                            preferred_element_type=F32) + b1_ref[...], 0.0)
    t = jnp.dot(t, w2_ref[...], preferred_element_type=F32) + b2_ref[...]
    t_ref[...] = t
    ssum[...] += jnp.sum(t, axis=0, keepdims=True)
    ssq[...] += jnp.sum(t * t, axis=0, keepdims=True)

    @pl.when(i == _NT - 1)
    def _():
        stats_ref[0:1, :] = ssum[...]
        stats_ref[1:2, :] = ssq[...]


def _node_layer(h, p0, p1, w1, b1, w2, b2):
    return pl.pallas_call(
        _node_layer_kernel,
        grid=(_NT,),
        in_specs=[
            pl.BlockSpec((_NROW, H), lambda i: (i, 0)),
            pl.BlockSpec((_NROW, H), lambda i: (i, 0)),
            pl.BlockSpec((_NROW, H), lambda i: (i, 0)),
            pl.BlockSpec((H, H), lambda i: (0, 0)),
            pl.BlockSpec((1, H), lambda i: (0, 0)),
            pl.BlockSpec((H, H), lambda i: (0, 0)),
            pl.BlockSpec((1, H), lambda i: (0, 0)),
        ],
        out_specs=[
            pl.BlockSpec((_NROW, H), lambda i: (i, 0)),
            pl.BlockSpec((2, H), lambda i: (0, 0)),
        ],
        out_shape=[
            jax.ShapeDtypeStruct((N, H), F32),
            jax.ShapeDtypeStruct((2, H), F32),
        ],
        scratch_shapes=[
            pltpu.VMEM((1, H), F32),
            pltpu.VMEM((1, H), F32),
        ],
    )(h, p0, p1, w1, b1.reshape(1, H), w2, b2.reshape(1, H))


def _bn_relu_kernel(t_ref, stats_ref, g_ref, b_ref, o_ref):
    mu = stats_ref[0:1, :] * (1.0 / N)
    var = stats_ref[1:2, :] * (1.0 / N) - mu * mu
    inv = lax.rsqrt(var + 1e-5)
    o_ref[...] = jnp.maximum(
        g_ref[...] * (t_ref[...] - mu) * inv + b_ref[...], 0.0)


def _bn_relu(t, stats, g, b):
    return pl.pallas_call(
        _bn_relu_kernel,
        grid=(_NT,),
        in_specs=[
            pl.BlockSpec((_NROW, H), lambda i: (i, 0)),
            pl.BlockSpec((2, H), lambda i: (0, 0)),
            pl.BlockSpec((1, H), lambda i: (0, 0)),
            pl.BlockSpec((1, H), lambda i: (0, 0)),
        ],
        out_specs=pl.BlockSpec((_NROW, H), lambda i: (i, 0)),
        out_shape=jax.ShapeDtypeStruct((N, H), F32),
    )(t, stats, g.reshape(1, H), b.reshape(1, H))


def _final_kernel(h_ref, batch_ref, ext_ref, wx1_ref, bx1_ref, wx2_ref,
                  bx2_ref, wf1_ref, bf1_ref, wf2_ref, bf2_ref,
                  o_ref, sums, cnts):
    i = pl.program_id(0)

    @pl.when(i == 0)
    def _():
        sums[...] = jnp.zeros_like(sums)
        cnts[...] = jnp.zeros_like(cnts)

    b = batch_ref[0]
    bb = jnp.broadcast_to(b, (G, _NROW))
    gi = lax.broadcasted_iota(jnp.int32, (G, _NROW), 0)
    oh = (bb == gi).astype(F32)
    sums[...] += jnp.dot(oh, h_ref[...], preferred_element_type=F32)
    cnts[...] += jnp.dot(oh, jnp.ones((_NROW, H), F32),
                         preferred_element_type=F32)

    @pl.when(i == _NT - 1)
    def _():
        emb = sums[...] / jnp.maximum(cnts[...], 1.0)
        ext = jnp.maximum(jnp.dot(ext_ref[...], wx1_ref[...],
                                  preferred_element_type=F32)
                          + bx1_ref[...], 0.0)
        ext = jnp.dot(ext, wx2_ref[...],
                      preferred_element_type=F32) + bx2_ref[...]
        comb = jnp.concatenate([emb, ext], axis=1)
        r = jnp.maximum(jnp.dot(comb, wf1_ref[...],
                                preferred_element_type=F32)
                        + bf1_ref[...], 0.0)
        o_ref[...] = jnp.dot(r, wf2_ref[...],
                             preferred_element_type=F32) + bf2_ref[...]


def _final(h, batch, ext, wx1, bx1, wx2, bx2, wf1, bf1, wf2, bf2):
    d = ext.shape[1]
    batch3 = batch.reshape(_NT, 1, _NROW)
    return pl.pallas_call(
        _final_kernel,
        grid=(_NT,),
        in_specs=[
            pl.BlockSpec((_NROW, H), lambda i: (i, 0)),
            pl.BlockSpec((1, 1, _NROW), lambda i: (i, 0, 0)),
            pl.BlockSpec((G, d), lambda i: (0, 0)),
            pl.BlockSpec((d, H), lambda i: (0, 0)),
            pl.BlockSpec((1, H), lambda i: (0, 0)),
            pl.BlockSpec((H, H), lambda i: (0, 0)),
            pl.BlockSpec((1, H), lambda i: (0, 0)),
            pl.BlockSpec((2 * H, H), lambda i: (0, 0)),
            pl.BlockSpec((1, H), lambda i: (0, 0)),
            pl.BlockSpec((H, 1), lambda i: (0, 0)),
            pl.BlockSpec((1, 1), lambda i: (0, 0)),
        ],
        out_specs=pl.BlockSpec((G, 1), lambda i: (0, 0)),
        out_shape=jax.ShapeDtypeStruct((G, 1), F32),
        scratch_shapes=[
            pltpu.VMEM((G, H), F32),
            pltpu.VMEM((G, H), F32),
        ],
    )(h, batch3, ext, wx1, bx1.reshape(1, H), wx2, bx2.reshape(1, H),
      wf1, bf1.reshape(1, H), wf2, bf2.reshape(1, 1))


def _edge_stage(h, e, src, dst):
    mesh = plsc.VectorSubcoreMesh(core_axis_name="c", subcore_axis_name="s")

    @functools.partial(
        pl.kernel,
        out_type=jax.ShapeDtypeStruct((2, N, H), F32),
        mesh=mesh,
        scratch_types=[
            pltpu.VMEM((_CHUNK,), jnp.int32),
            pltpu.VMEM((_CHUNK,), jnp.int32),
            pltpu.VMEM((_CHUNK,), jnp.int32),
            pltpu.VMEM((_CHUNK,), jnp.int32),
            pltpu.VMEM((_CHUNK,), jnp.int32),
            pltpu.VMEM((_CHUNK,), jnp.int32),
            pltpu.VMEM((_CHUNK,), jnp.int32),
            pltpu.VMEM((_CHUNK,), jnp.int32),
            pltpu.VMEM((_CHUNK, H), F32),
            pltpu.VMEM((_CHUNK, H), F32),
            pltpu.VMEM((_CHUNK, H), F32),
            pltpu.VMEM((_CHUNK, H), F32),
            pltpu.VMEM_SHARED((_NPAD, H), F32),
            pltpu.SemaphoreType.DMA,
            pltpu.SemaphoreType.DMA,
            pltpu.SemaphoreType.DMA,
            pltpu.SemaphoreType.DMA,
            pltpu.SemaphoreType.DMA,
            pltpu.SemaphoreType.DMA,
            pltpu.SemaphoreType.DMA,
            pltpu.SemaphoreType.DMA,
        ],
    )
    def k(h_hbm, e_hbm, src_hbm, dst_hbm, out_hbm,
          sc0, sc1, sc2, sc3, dc0, dc1, dc2, dc3, r0, r1, e0, e1,
          acc, g0, g1, s0, s1, x0, x1, x2, x3):
        cid = lax.axis_index("c")
        sid = lax.axis_index("s")
        w = cid * 16 + sid
        srcb = (sc0, sc1, sc2, sc3)
        dstb = (dc0, dc1, dc2, dc3)
        rows = (r0, r1)
        ebuf = (e0, e1)
        gsem = (g0, g1)
        esem = (s0, s1)
        xsem = (x0, x1, x2, x3)
        base0 = w * _CPT * _CHUNK

        def issue_idx(i, q):
            pltpu.async_copy(src_hbm.at[pl.ds(base0 + i * _CHUNK, _CHUNK)],
                             srcb[q], xsem[q])
            pltpu.async_copy(dst_hbm.at[pl.ds(base0 + i * _CHUNK, _CHUNK)],
                             dstb[q], xsem[q])

        def wait_idx(i, q):
            pltpu.make_async_copy(src_hbm.at[pl.ds(base0 + i * _CHUNK,
                                                   _CHUNK)],
                                  srcb[q], xsem[q]).wait()
            pltpu.make_async_copy(dst_hbm.at[pl.ds(base0 + i * _CHUNK,
                                                   _CHUNK)],
                                  dstb[q], xsem[q]).wait()

        def issue_data(i, b, q):
            pltpu.async_copy(h_hbm.at[srcb[q]], rows[b], gsem[b])
            pltpu.async_copy(e_hbm.at[pl.ds(base0 + i * _CHUNK, _CHUNK)],
                             ebuf[b], esem[b])

        def wait_data(i, b, q):
            pltpu.make_async_copy(h_hbm.at[srcb[q]], rows[b], gsem[b]).wait()
            pltpu.make_async_copy(e_hbm.at[pl.ds(base0 + i * _CHUNK, _CHUNK)],
                                  ebuf[b], esem[b]).wait()

        for q in range(4):
            issue_idx(q, q)
        wait_idx(0, 0)
        issue_data(0, 0, 0)

        @pl.loop(0, _CHUNK)
        def _(r):
            for j in range(H // 16):
                r1[r, pl.ds(j * 16, 16)] = jnp.zeros((16,), F32)
        row0 = sid * _RPT
        for t in range(_RPT // _CHUNK):
            pltpu.sync_copy(r1, acc.at[pl.ds(row0 + t * _CHUNK, _CHUNK)])
        rem = _RPT % _CHUNK
        if rem:
            pltpu.sync_copy(r1.at[pl.ds(0, rem)],
                            acc.at[pl.ds(row0 + _RPT - rem, rem)])

        @pl.when(sid == 15)
        def _():
            pltpu.sync_copy(r1.at[pl.ds(0, _NPAD - 16 * _RPT)],
                            acc.at[pl.ds(16 * _RPT, _NPAD - 16 * _RPT)])
        plsc.subcore_barrier()

        def step(i, b, q):
            qn = (q + 1) % 4

            @pl.when(i + 1 < _CPT)
            def _():
                wait_idx(i + 1, qn)
                issue_data(i + 1, 1 - b, qn)
            wait_data(i, b, q)
            rb = rows[b]
            eb = ebuf[b]

            @pl.loop(0, _CHUNK)
            def _(r):
                for j in range(H // 16):
                    sl = pl.ds(j * 16, 16)
                    rb[r, sl] = jnp.maximum(rb[r, sl] + eb[r, sl], 0.0)

            pltpu.sync_copy(rb, acc.at[dstb[q]], add=True)

            @pl.when(i + 4 < _CPT)
            def _():
                issue_idx(i + 4, q)

        @pl.loop(0, _CPT, step=4)
        def _(i):
            for j in range(4):
                step(i + j, j % 2, j)

        plsc.subcore_barrier()
        pltpu.sync_copy(acc.at[pl.ds(row0, _RPT)],
                        out_hbm.at[cid].at[pl.ds(row0, _RPT)])

        @pl.when(sid == 15)
        def _():
            pltpu.sync_copy(acc.at[pl.ds(16 * _RPT, N - 16 * _RPT)],
                            out_hbm.at[cid].at[pl.ds(16 * _RPT, N - 16 * _RPT)])

    return k(h, e, src, dst)


def kernel(x, edge_index, edge_attr, batch, externals, W_node, b_node,
           We1, be1, We2, be2, Wc1, bc1, Wc2, bc2, gamma, beta,
           Wx1, bx1, Wx2, bx2, Wf1, bf1, Wf2, bf2):
    npad = _EPAD - E
    src_p = jnp.concatenate([edge_index[0], jnp.zeros((npad,), jnp.int32)])
    dst_p = jnp.concatenate([edge_index[1], jnp.full((npad,), N, jnp.int32)])
    ea_pad = jnp.concatenate(
        [edge_attr, jnp.zeros((npad, edge_attr.shape[1]), F32)])
    h = _node_embed(x, W_node, b_node)
    e = _edge_mlp(ea_pad, We1, be1, We2, be2)
    for l in range(Wc1.shape[0]):
        parts = _edge_stage(h, e, src_p, dst_p)
        t, stats = _node_layer(h, parts[0], parts[1],
                               Wc1[l], bc1[l], Wc2[l], bc2[l])
        h = _bn_relu(t, stats, gamma[l], beta[l])
    out = _final(h, batch, externals,
                 Wx1, bx1, Wx2, bx2, Wf1, bf1, Wf2, bf2)
    return out[:, 0]


# speedup vs baseline: 2.1908x; 2.1908x over previous
import functools

import jax
import jax.numpy as jnp
from jax import lax
from jax.experimental import pallas as pl
from jax.experimental.pallas import tpu as pltpu
from jax.experimental.pallas import tpu_sc as plsc

N = 10000
E = 320000
G = 256
H = 128
F32 = jnp.float32

_NT = 5
_NROW = N // _NT
_EROW = 2560

_CHUNK = 64
_CPT = 156
_RPT = 624


def _mm_bias_kernel(x_ref, w_ref, b_ref, o_ref):
    o_ref[...] = jnp.dot(x_ref[...], w_ref[...],
                         preferred_element_type=F32) + b_ref[...]


def _node_embed(x, w, b):
    return pl.pallas_call(
        _mm_bias_kernel,
        grid=(_NT,),
        in_specs=[
            pl.BlockSpec((_NROW, H), lambda i: (i, 0)),
            pl.BlockSpec((H, H), lambda i: (0, 0)),
            pl.BlockSpec((1, H), lambda i: (0, 0)),
        ],
        out_specs=pl.BlockSpec((_NROW, H), lambda i: (i, 0)),
        out_shape=jax.ShapeDtypeStruct((N, H), F32),
    )(x, w, b.reshape(1, H))


def _edge_mlp_kernel(a_ref, w1_ref, b1_ref, w2_ref, b2_ref, o_ref):
    t = jnp.maximum(jnp.dot(a_ref[...], w1_ref[...],
                            preferred_element_type=F32) + b1_ref[...], 0.0)
    o_ref[...] = jnp.dot(t, w2_ref[...],
                         preferred_element_type=F32) + b2_ref[...]


def _edge_mlp(a, w1, b1, w2, b2):
    d = a.shape[1]
    ne = a.shape[0]
    return pl.pallas_call(
        _edge_mlp_kernel,
        grid=(ne // _EROW,),
        in_specs=[
            pl.BlockSpec((_EROW, d), lambda i: (i, 0)),
            pl.BlockSpec((d, H), lambda i: (0, 0)),
            pl.BlockSpec((1, H), lambda i: (0, 0)),
            pl.BlockSpec((H, H), lambda i: (0, 0)),
            pl.BlockSpec((1, H), lambda i: (0, 0)),
        ],
        out_specs=pl.BlockSpec((_EROW, H), lambda i: (i, 0)),
        out_shape=jax.ShapeDtypeStruct((ne, H), F32),
    )(a, w1, b1.reshape(1, H), w2, b2.reshape(1, H))


def _node_layer_kernel(h_ref, p0_ref, p1_ref, w1_ref, b1_ref, w2_ref, b2_ref,
                       t_ref, stats_ref, ssum, ssq):
    i = pl.program_id(0)

    @pl.when(i == 0)
    def _():
        ssum[...] = jnp.zeros_like(ssum)
        ssq[...] = jnp.zeros_like(ssq)

    z = h_ref[...] + p0_ref[...] + p1_ref[...]
    t = jnp.maximum(jnp.dot(z, w1_ref[...],
                            preferred_element_type=F32) + b1_ref[...], 0.0)
    t = jnp.dot(t, w2_ref[...], preferred_element_type=F32) + b2_ref[...]
    t_ref[...] = t
    ssum[...] += jnp.sum(t, axis=0, keepdims=True)
    ssq[...] += jnp.sum(t * t, axis=0, keepdims=True)

    @pl.when(i == _NT - 1)
    def _():
        stats_ref[0:1, :] = ssum[...]
        stats_ref[1:2, :] = ssq[...]


def _node_layer(h, p0, p1, w1, b1, w2, b2):
    return pl.pallas_call(
        _node_layer_kernel,
        grid=(_NT,),
        in_specs=[
            pl.BlockSpec((_NROW, H), lambda i: (i, 0)),
            pl.BlockSpec((_NROW, H), lambda i: (i, 0)),
            pl.BlockSpec((_NROW, H), lambda i: (i, 0)),
            pl.BlockSpec((H, H), lambda i: (0, 0)),
            pl.BlockSpec((1, H), lambda i: (0, 0)),
            pl.BlockSpec((H, H), lambda i: (0, 0)),
            pl.BlockSpec((1, H), lambda i: (0, 0)),
        ],
        out_specs=[
            pl.BlockSpec((_NROW, H), lambda i: (i, 0)),
            pl.BlockSpec((2, H), lambda i: (0, 0)),
        ],
        out_shape=[
            jax.ShapeDtypeStruct((N, H), F32),
            jax.ShapeDtypeStruct((2, H), F32),
        ],
        scratch_shapes=[
            pltpu.VMEM((1, H), F32),
            pltpu.VMEM((1, H), F32),
        ],
    )(h, p0, p1, w1, b1.reshape(1, H), w2, b2.reshape(1, H))


def _bn_relu_kernel(t_ref, stats_ref, g_ref, b_ref, o_ref):
    mu = stats_ref[0:1, :] * (1.0 / N)
    var = stats_ref[1:2, :] * (1.0 / N) - mu * mu
    inv = lax.rsqrt(var + 1e-5)
    o_ref[...] = jnp.maximum(
        g_ref[...] * (t_ref[...] - mu) * inv + b_ref[...], 0.0)


def _bn_relu(t, stats, g, b):
    return pl.pallas_call(
        _bn_relu_kernel,
        grid=(_NT,),
        in_specs=[
            pl.BlockSpec((_NROW, H), lambda i: (i, 0)),
            pl.BlockSpec((2, H), lambda i: (0, 0)),
            pl.BlockSpec((1, H), lambda i: (0, 0)),
            pl.BlockSpec((1, H), lambda i: (0, 0)),
        ],
        out_specs=pl.BlockSpec((_NROW, H), lambda i: (i, 0)),
        out_shape=jax.ShapeDtypeStruct((N, H), F32),
    )(t, stats, g.reshape(1, H), b.reshape(1, H))


def _final_kernel(h_ref, batch_ref, ext_ref, wx1_ref, bx1_ref, wx2_ref,
                  bx2_ref, wf1_ref, bf1_ref, wf2_ref, bf2_ref,
                  o_ref, sums, cnts):
    i = pl.program_id(0)

    @pl.when(i == 0)
    def _():
        sums[...] = jnp.zeros_like(sums)
        cnts[...] = jnp.zeros_like(cnts)

    b = batch_ref[0]
    bb = jnp.broadcast_to(b, (G, _NROW))
    gi = lax.broadcasted_iota(jnp.int32, (G, _NROW), 0)
    oh = (bb == gi).astype(F32)
    sums[...] += jnp.dot(oh, h_ref[...], preferred_element_type=F32)
    cnts[...] += jnp.dot(oh, jnp.ones((_NROW, H), F32),
                         preferred_element_type=F32)

    @pl.when(i == _NT - 1)
    def _():
        emb = sums[...] / jnp.maximum(cnts[...], 1.0)
        ext = jnp.maximum(jnp.dot(ext_ref[...], wx1_ref[...],
                                  preferred_element_type=F32)
                          + bx1_ref[...], 0.0)
        ext = jnp.dot(ext, wx2_ref[...],
                      preferred_element_type=F32) + bx2_ref[...]
        comb = jnp.concatenate([emb, ext], axis=1)
        r = jnp.maximum(jnp.dot(comb, wf1_ref[...],
                                preferred_element_type=F32)
                        + bf1_ref[...], 0.0)
        o_ref[...] = jnp.dot(r, wf2_ref[...],
                             preferred_element_type=F32) + bf2_ref[...]


def _final(h, batch, ext, wx1, bx1, wx2, bx2, wf1, bf1, wf2, bf2):
    d = ext.shape[1]
    batch3 = batch.reshape(_NT, 1, _NROW)
    return pl.pallas_call(
        _final_kernel,
        grid=(_NT,),
        in_specs=[
            pl.BlockSpec((_NROW, H), lambda i: (i, 0)),
            pl.BlockSpec((1, 1, _NROW), lambda i: (i, 0, 0)),
            pl.BlockSpec((G, d), lambda i: (0, 0)),
            pl.BlockSpec((d, H), lambda i: (0, 0)),
            pl.BlockSpec((1, H), lambda i: (0, 0)),
            pl.BlockSpec((H, H), lambda i: (0, 0)),
            pl.BlockSpec((1, H), lambda i: (0, 0)),
            pl.BlockSpec((2 * H, H), lambda i: (0, 0)),
            pl.BlockSpec((1, H), lambda i: (0, 0)),
            pl.BlockSpec((H, 1), lambda i: (0, 0)),
            pl.BlockSpec((1, 1), lambda i: (0, 0)),
        ],
        out_specs=pl.BlockSpec((G, 1), lambda i: (0, 0)),
        out_shape=jax.ShapeDtypeStruct((G, 1), F32),
        scratch_shapes=[
            pltpu.VMEM((G, H), F32),
            pltpu.VMEM((G, H), F32),
        ],
    )(h, batch3, ext, wx1, bx1.reshape(1, H), wx2, bx2.reshape(1, H),
      wf1, bf1.reshape(1, H), wf2, bf2.reshape(1, 1))


def _edge_stage(h, e, src, dst):
    mesh = plsc.VectorSubcoreMesh(core_axis_name="c", subcore_axis_name="s")

    @functools.partial(
        pl.kernel,
        out_type=jax.ShapeDtypeStruct((2, N, H), F32),
        mesh=mesh,
        scratch_types=[
            pltpu.VMEM((_CHUNK,), jnp.int32),
            pltpu.VMEM((_CHUNK,), jnp.int32),
            pltpu.VMEM((_CHUNK,), jnp.int32),
            pltpu.VMEM((_CHUNK,), jnp.int32),
            pltpu.VMEM((_CHUNK,), jnp.int32),
            pltpu.VMEM((_CHUNK,), jnp.int32),
            pltpu.VMEM((_CHUNK,), jnp.int32),
            pltpu.VMEM((_CHUNK,), jnp.int32),
            pltpu.VMEM((_CHUNK, H), F32),
            pltpu.VMEM((_CHUNK, H), F32),
            pltpu.VMEM((_CHUNK, H), F32),
            pltpu.VMEM((_CHUNK, H), F32),
            pltpu.VMEM_SHARED((N, H), F32),
            pltpu.SemaphoreType.DMA,
            pltpu.SemaphoreType.DMA,
            pltpu.SemaphoreType.DMA,
            pltpu.SemaphoreType.DMA,
            pltpu.SemaphoreType.DMA,
            pltpu.SemaphoreType.DMA,
            pltpu.SemaphoreType.DMA,
            pltpu.SemaphoreType.DMA,
        ],
    )
    def k(h_hbm, e_hbm, src_hbm, dst_hbm, out_hbm,
          sc0, sc1, sc2, sc3, dc0, dc1, dc2, dc3, r0, r1, e0, e1,
          acc, g0, g1, s0, s1, x0, x1, x2, x3):
        cid = lax.axis_index("c")
        sid = lax.axis_index("s")
        w = cid * 16 + sid
        srcb = (sc0, sc1, sc2, sc3)
        dstb = (dc0, dc1, dc2, dc3)
        rows = (r0, r1)
        ebuf = (e0, e1)
        gsem = (g0, g1)
        esem = (s0, s1)
        xsem = (x0, x1, x2, x3)
        base0 = (w * _CPT + jnp.minimum(w, 8)) * _CHUNK

        def issue_idx(i, q):
            pltpu.async_copy(src_hbm.at[pl.ds(base0 + i * _CHUNK, _CHUNK)],
                             srcb[q], xsem[q])
            pltpu.async_copy(dst_hbm.at[pl.ds(base0 + i * _CHUNK, _CHUNK)],
                             dstb[q], xsem[q])

        def wait_idx(i, q):
            pltpu.make_async_copy(src_hbm.at[pl.ds(base0 + i * _CHUNK,
                                                   _CHUNK)],
                                  srcb[q], xsem[q]).wait()
            pltpu.make_async_copy(dst_hbm.at[pl.ds(base0 + i * _CHUNK,
                                                   _CHUNK)],
                                  dstb[q], xsem[q]).wait()

        def issue_data(i, b, q):
            pltpu.async_copy(h_hbm.at[srcb[q]], rows[b], gsem[b])
            pltpu.async_copy(e_hbm.at[pl.ds(base0 + i * _CHUNK, _CHUNK)],
                             ebuf[b], esem[b])

        def wait_data(i, b, q):
            pltpu.make_async_copy(h_hbm.at[srcb[q]], rows[b], gsem[b]).wait()
            pltpu.make_async_copy(e_hbm.at[pl.ds(base0 + i * _CHUNK, _CHUNK)],
                                  ebuf[b], esem[b]).wait()

        for q in range(4):
            issue_idx(q, q)
        wait_idx(0, 0)
        issue_data(0, 0, 0)

        @pl.loop(0, _CHUNK)
        def _(r):
            for j in range(H // 16):
                r1[r, pl.ds(j * 16, 16)] = jnp.zeros((16,), F32)
        row0 = sid * _RPT
        for t in range(_RPT // _CHUNK):
            pltpu.sync_copy(r1, acc.at[pl.ds(row0 + t * _CHUNK, _CHUNK)])
        rem = _RPT % _CHUNK
        if rem:
            pltpu.sync_copy(r1.at[pl.ds(0, rem)],
                            acc.at[pl.ds(row0 + _RPT - rem, rem)])

        @pl.when(sid == 15)
        def _():
            pltpu.sync_copy(r1.at[pl.ds(0, N - 16 * _RPT)],
                            acc.at[pl.ds(16 * _RPT, N - 16 * _RPT)])
        plsc.subcore_barrier()

        def step(i, b, q):
            qn = (q + 1) % 4

            @pl.when(i + 1 < _CPT)
            def _():
                wait_idx(i + 1, qn)
                issue_data(i + 1, 1 - b, qn)
            wait_data(i, b, q)
            rb = rows[b]
            eb = ebuf[b]

            @pl.loop(0, _CHUNK)
            def _(r):
                for j in range(H // 16):
                    sl = pl.ds(j * 16, 16)
                    rb[r, sl] = jnp.maximum(rb[r, sl] + eb[r, sl], 0.0)

            pltpu.sync_copy(rb, acc.at[dstb[q]], add=True)

            @pl.when(i + 4 < _CPT)
            def _():
                issue_idx(i + 4, q)

        @pl.loop(0, _CPT, step=4)
        def _(i):
            for j in range(4):
                step(i + j, j % 2, j)

        @pl.when(w < 8)
        def _():
            eb0 = base0 + _CPT * _CHUNK
            pltpu.sync_copy(src_hbm.at[pl.ds(eb0, _CHUNK)], sc0)
            pltpu.sync_copy(dst_hbm.at[pl.ds(eb0, _CHUNK)], dc0)
            pltpu.async_copy(h_hbm.at[sc0], r0, g0).wait()
            pltpu.sync_copy(e_hbm.at[pl.ds(eb0, _CHUNK)], e0)

            @pl.loop(0, _CHUNK)
            def _(r):
                for j in range(H // 16):
                    sl = pl.ds(j * 16, 16)
                    r0[r, sl] = jnp.maximum(r0[r, sl] + e0[r, sl], 0.0)

            pltpu.sync_copy(r0, acc.at[dc0], add=True)

        plsc.subcore_barrier()
        pltpu.sync_copy(acc.at[pl.ds(row0, _RPT)],
                        out_hbm.at[cid].at[pl.ds(row0, _RPT)])

        @pl.when(sid == 15)
        def _():
            pltpu.sync_copy(acc.at[pl.ds(16 * _RPT, N - 16 * _RPT)],
                            out_hbm.at[cid].at[pl.ds(16 * _RPT, N - 16 * _RPT)])

    return k(h, e, src, dst)


def kernel(x, edge_index, edge_attr, batch, externals, W_node, b_node,
           We1, be1, We2, be2, Wc1, bc1, Wc2, bc2, gamma, beta,
           Wx1, bx1, Wx2, bx2, Wf1, bf1, Wf2, bf2):
    src = edge_index[0]
    dst = edge_index[1]
    h = _node_embed(x, W_node, b_node)
    e = _edge_mlp(edge_attr, We1, be1, We2, be2)
    for l in range(Wc1.shape[0]):
        parts = _edge_stage(h, e, src, dst)
        t, stats = _node_layer(h, parts[0], parts[1],
                               Wc1[l], bc1[l], Wc2[l], bc2[l])
        h = _bn_relu(t, stats, gamma[l], beta[l])
    out = _final(h, batch, externals,
                 Wx1, bx1, Wx2, bx2, Wf1, bf1, Wf2, bf2)
    return out[:, 0]
